# baseline (speedup 1.0000x reference)
import jax
import jax.numpy as jnp
import numpy as np
from jax.experimental import pallas as pl
from jax.experimental.pallas import tpu as pltpu
from jax.sharding import Mesh, NamedSharding, PartitionSpec as P

_B, _C, _H, _W = 4, 64, 256, 512
_HB = 8


def _resample_body(x2_ref, x1_ref, o_ref, m_scr, o_scr):
    m_scr[...] = jnp.swapaxes(
        x1_ref[0, :, 0, :, :], 0, 1).reshape(_HB * _C, _W)
    disp = x2_ref[0, 0, :, :]
    iota_w = jax.lax.broadcasted_iota(
        jnp.int32, (_HB, _W), 1).astype(jnp.float32)
    x = iota_w + disp
    valid = (x >= 0.0) & (x <= float(_W - 1))
    xa = jnp.where(valid, x, -2.0)
    col = jax.lax.broadcasted_iota(
        jnp.int32, (_W, _W), 0).astype(jnp.float32)
    for hi in range(_HB):
        xr = xa[hi:hi + 1, :]
        s = 1.0 - jnp.minimum(jnp.abs(col - xr), 1.0)
        lhs = m_scr[hi * _C:(hi + 1) * _C, :]
        o_scr[hi * _C:(hi + 1) * _C, :] = jnp.dot(
            lhs, s, preferred_element_type=jnp.float32)
    o_ref[0, :, 0, :, :] = jnp.swapaxes(
        o_scr[...].reshape(_HB, _C, _W), 0, 1)


def _resample(input1, input2):
    b, c, h, w = input1.shape
    x1 = input1.reshape(b, c, h // _HB, _HB, w)
    x2 = input2.reshape(b, h // _HB, _HB, w)
    out = pl.pallas_call(
        _resample_body,
        grid=(b, h // _HB),
        in_specs=[
            pl.BlockSpec((1, 1, _HB, w), lambda bi, hb: (bi, hb, 0, 0)),
            pl.BlockSpec((1, c, 1, _HB, w), lambda bi, hb: (bi, 0, hb, 0, 0)),
        ],
        out_specs=pl.BlockSpec(
            (1, c, 1, _HB, w), lambda bi, hb: (bi, 0, hb, 0, 0)),
        out_shape=jax.ShapeDtypeStruct((b, c, h // _HB, _HB, w), jnp.float32),
        scratch_shapes=[
            pltpu.VMEM((_HB * c, w), jnp.float32),
            pltpu.VMEM((_HB * c, w), jnp.float32),
        ],
        compiler_params=pltpu.CompilerParams(
            dimension_semantics=("parallel", "arbitrary"),
            vmem_limit_bytes=56 * 1024 * 1024,
        ),
    )(x2, x1)
    return out.reshape(b, c, h, w)


def kernel(input1, input2):
    devs = jax.devices()
    if len(devs) < 2:
        return _resample(input1, input2)
    mesh = Mesh(np.array(devs[:2]), ("d",))
    sh = NamedSharding(mesh, P("d"))
    x1 = jax.device_put(input1, sh)
    x2 = jax.device_put(input2, sh)
    f = jax.shard_map(_resample, mesh=mesh, in_specs=(P("d"), P("d")),
                      out_specs=P("d"), check_vma=False)
    return f(x1, x2)


# speedup vs baseline: 2.9021x; 2.9021x over previous
import jax
import jax.numpy as jnp
import numpy as np
from jax.experimental import pallas as pl
from jax.experimental.pallas import tpu as pltpu
from jax.sharding import Mesh, NamedSharding, PartitionSpec as P

_B, _C, _H, _W = 4, 64, 256, 512
_HB = 8


def _resample_body(x2_ref, x1_ref, o_ref, m_scr, o_scr):
    m_scr[...] = jnp.swapaxes(
        x1_ref[0, :, 0, :, :], 0, 1).reshape(_HB * _C, _W)
    disp = x2_ref[0, 0, :, :]
    iota_w = jax.lax.broadcasted_iota(
        jnp.int32, (_HB, _W), 1).astype(jnp.float32)
    x = iota_w + disp
    valid = (x >= 0.0) & (x <= float(_W - 1))
    xa = jnp.where(valid, x, -2.0)
    col = jax.lax.broadcasted_iota(
        jnp.int32, (_W, _W), 0).astype(jnp.float32)
    for hi in range(_HB):
        xr = xa[hi:hi + 1, :]
        s = 1.0 - jnp.minimum(jnp.abs(col - xr), 1.0)
        lhs = m_scr[hi * _C:(hi + 1) * _C, :]
        o_scr[hi * _C:(hi + 1) * _C, :] = jnp.dot(
            lhs, s, preferred_element_type=jnp.float32)
    o_ref[0, :, 0, :, :] = jnp.swapaxes(
        o_scr[...].reshape(_HB, _C, _W), 0, 1)


def _resample(input1, input2):
    b, c, h, w = input1.shape
    x1 = input1.reshape(b, c, h // _HB, _HB, w)
    x2 = input2.reshape(b, h // _HB, _HB, w)
    out = pl.pallas_call(
        _resample_body,
        grid=(b, h // _HB),
        in_specs=[
            pl.BlockSpec((1, 1, _HB, w), lambda bi, hb: (bi, hb, 0, 0)),
            pl.BlockSpec((1, c, 1, _HB, w), lambda bi, hb: (bi, 0, hb, 0, 0)),
        ],
        out_specs=pl.BlockSpec(
            (1, c, 1, _HB, w), lambda bi, hb: (bi, 0, hb, 0, 0)),
        out_shape=jax.ShapeDtypeStruct((b, c, h // _HB, _HB, w), jnp.float32),
        scratch_shapes=[
            pltpu.VMEM((_HB * c, w), jnp.float32),
            pltpu.VMEM((_HB * c, w), jnp.float32),
        ],
        compiler_params=pltpu.CompilerParams(
            dimension_semantics=("parallel", "arbitrary"),
            vmem_limit_bytes=56 * 1024 * 1024,
        ),
    )(x2, x1)
    return out.reshape(b, c, h, w)


def kernel(input1, input2):
    return _resample(input1, input2)


# speedup vs baseline: 3.6672x; 1.2636x over previous
import jax
import jax.numpy as jnp
import numpy as np
from jax.experimental import pallas as pl
from jax.experimental.pallas import tpu as pltpu
from jax.sharding import Mesh, NamedSharding, PartitionSpec as P

_B, _C, _H, _W = 4, 64, 256, 512
_HB = 16


def _resample_body(x2_ref, x1_ref, o_ref, m_scr, o_scr):
    m_scr[...] = jnp.swapaxes(
        x1_ref[0, :, 0, :, :], 0, 1).reshape(_HB * _C, _W).astype(
            jnp.bfloat16)
    disp = x2_ref[0, 0, :, :]
    iota_w = jax.lax.broadcasted_iota(
        jnp.int32, (_HB, _W), 1).astype(jnp.float32)
    x = iota_w + disp
    valid = (x >= 0.0) & (x <= float(_W - 1))
    xa = jnp.where(valid, x, -2.0)
    col = jax.lax.broadcasted_iota(
        jnp.int32, (_W, _W), 0).astype(jnp.float32)
    one = jnp.bfloat16(1.0)
    for hi in range(_HB):
        xr = xa[hi:hi + 1, :]
        t = (col - xr).astype(jnp.bfloat16)
        s = one - jnp.minimum(jnp.abs(t), one)
        lhs = m_scr[hi * _C:(hi + 1) * _C, :]
        o_scr[hi * _C:(hi + 1) * _C, :] = jnp.dot(
            lhs, s, preferred_element_type=jnp.float32)
    o_ref[0, :, 0, :, :] = jnp.swapaxes(
        o_scr[...].reshape(_HB, _C, _W), 0, 1)


def _resample(input1, input2):
    b, c, h, w = input1.shape
    x1 = input1.reshape(b, c, h // _HB, _HB, w)
    x2 = input2.reshape(b, h // _HB, _HB, w)
    out = pl.pallas_call(
        _resample_body,
        grid=(b, h // _HB),
        in_specs=[
            pl.BlockSpec((1, 1, _HB, w), lambda bi, hb: (bi, hb, 0, 0)),
            pl.BlockSpec((1, c, 1, _HB, w), lambda bi, hb: (bi, 0, hb, 0, 0)),
        ],
        out_specs=pl.BlockSpec(
            (1, c, 1, _HB, w), lambda bi, hb: (bi, 0, hb, 0, 0)),
        out_shape=jax.ShapeDtypeStruct((b, c, h // _HB, _HB, w), jnp.float32),
        scratch_shapes=[
            pltpu.VMEM((_HB * c, w), jnp.bfloat16),
            pltpu.VMEM((_HB * c, w), jnp.float32),
        ],
        compiler_params=pltpu.CompilerParams(
            dimension_semantics=("parallel", "arbitrary"),
            vmem_limit_bytes=56 * 1024 * 1024,
        ),
    )(x2, x1)
    return out.reshape(b, c, h, w)


def kernel(input1, input2):
    return _resample(input1, input2)


# speedup vs baseline: 4.1542x; 1.1328x over previous
import jax
import jax.numpy as jnp
import numpy as np
from jax.experimental import pallas as pl
from jax.experimental.pallas import tpu as pltpu
from jax.sharding import Mesh, NamedSharding, PartitionSpec as P

_B, _C, _H, _W = 4, 64, 256, 512
_HB = 16


_G = 65
_GO = 17
_HALF = _HB // 8


def _resample_body(x2_ref, x1_ref, o_ref, m_scr, o_scr):
    val = x1_ref[0, :, 0, :, :].reshape(_C * _HB, _W)
    for ci in range(_C):
        for half in range(_HALF):
            row0 = (ci * _HALF + half) * 8
            dst0 = half * 8 * _G + ci
            for wc in range(4):
                m_scr[wc, dst0:dst0 + 8 * _G:_G, :] = (
                    val[row0:row0 + 8, wc * 128:(wc + 1) * 128])
    disp = x2_ref[0, 0, :, :]
    iota_w = jax.lax.broadcasted_iota(
        jnp.int32, (_HB, _W), 1).astype(jnp.float32)
    x = iota_w + disp
    valid = (x >= 0.0) & (x <= float(_W - 1))
    xa = jnp.where(valid, x, -2.0)
    col = jax.lax.broadcasted_iota(
        jnp.int32, (_W, _W), 0).astype(jnp.float32)
    one = jnp.bfloat16(1.0)
    for hi in range(_HB):
        xr = xa[hi:hi + 1, :]
        t = (col - xr).astype(jnp.bfloat16)
        s = one - jnp.minimum(jnp.abs(t), one)
        base = hi * _G
        lhs = jnp.concatenate(
            [m_scr[wc, base:base + _C, :] for wc in range(4)],
            axis=-1).astype(jnp.bfloat16)
        res = jnp.dot(lhs, s, preferred_element_type=jnp.float32)
        for g in range(_C // 8):
            dst0 = 8 * g * _GO + hi
            for wc in range(4):
                o_scr[wc, dst0:dst0 + 8 * _GO:_GO, :] = (
                    res[8 * g:8 * g + 8, wc * 128:(wc + 1) * 128])
    for ci in range(_C):
        for wc in range(4):
            o_ref[0, ci, 0, :, wc * 128:(wc + 1) * 128] = (
                o_scr[wc, ci * _GO:ci * _GO + _HB, :])


def _resample(input1, input2):
    b, c, h, w = input1.shape
    x1 = input1.reshape(b, c, h // _HB, _HB, w)
    x2 = input2.reshape(b, h // _HB, _HB, w)
    out = pl.pallas_call(
        _resample_body,
        grid=(b, h // _HB),
        in_specs=[
            pl.BlockSpec((1, 1, _HB, w), lambda bi, hb: (bi, hb, 0, 0)),
            pl.BlockSpec((1, c, 1, _HB, w), lambda bi, hb: (bi, 0, hb, 0, 0)),
        ],
        out_specs=pl.BlockSpec(
            (1, c, 1, _HB, w), lambda bi, hb: (bi, 0, hb, 0, 0)),
        out_shape=jax.ShapeDtypeStruct((b, c, h // _HB, _HB, w), jnp.float32),
        scratch_shapes=[
            pltpu.VMEM((4, (_HB - 1) * _G + c + 1, 128), jnp.float32),
            pltpu.VMEM((4, (c - 1) * _GO + _HB + 1, 128), jnp.float32),
        ],
        compiler_params=pltpu.CompilerParams(
            dimension_semantics=("parallel", "arbitrary"),
            vmem_limit_bytes=56 * 1024 * 1024,
        ),
    )(x2, x1)
    return out.reshape(b, c, h, w)


def kernel(input1, input2):
    return _resample(input1, input2)


# speedup vs baseline: 4.1607x; 1.0016x over previous
import jax
import jax.numpy as jnp
import numpy as np
from jax.experimental import pallas as pl
from jax.experimental.pallas import tpu as pltpu
from jax.sharding import Mesh, NamedSharding, PartitionSpec as P

_B, _C, _H, _W = 4, 64, 256, 512
_HB = 32


_G = 65
_GO = 17
_HALF = _HB // 8


def _resample_body(x2_ref, x1_ref, o_ref, m_scr, o_scr):
    val = x1_ref[0, :, 0, :, :].reshape(_C * _HB, _W)
    for ci in range(_C):
        for half in range(_HALF):
            row0 = (ci * _HALF + half) * 8
            dst0 = half * 8 * _G + ci
            for wc in range(4):
                m_scr[wc, dst0:dst0 + 8 * _G:_G, :] = (
                    val[row0:row0 + 8, wc * 128:(wc + 1) * 128])
    disp = x2_ref[0, 0, :, :]
    iota_w = jax.lax.broadcasted_iota(
        jnp.int32, (_HB, _W), 1).astype(jnp.float32)
    x = iota_w + disp
    valid = (x >= 0.0) & (x <= float(_W - 1))
    xa = jnp.where(valid, x, -2.0)
    col = jax.lax.broadcasted_iota(
        jnp.int32, (_W, _W), 0).astype(jnp.float32)
    one = jnp.bfloat16(1.0)
    for hi in range(_HB):
        xr = xa[hi:hi + 1, :]
        t = (col - xr).astype(jnp.bfloat16)
        s = one - jnp.minimum(jnp.abs(t), one)
        base = hi * _G
        lhs = jnp.concatenate(
            [m_scr[wc, base:base + _C, :] for wc in range(4)],
            axis=-1).astype(jnp.bfloat16)
        res = jnp.dot(lhs, s, preferred_element_type=jnp.float32)
        for g in range(_C // 8):
            dst0 = 8 * g * _GO + hi
            for wc in range(4):
                o_scr[wc, dst0:dst0 + 8 * _GO:_GO, :] = (
                    res[8 * g:8 * g + 8, wc * 128:(wc + 1) * 128])
    for ci in range(_C):
        for wc in range(4):
            o_ref[0, ci, 0, :, wc * 128:(wc + 1) * 128] = (
                o_scr[wc, ci * _GO:ci * _GO + _HB, :])


def _resample(input1, input2):
    b, c, h, w = input1.shape
    x1 = input1.reshape(b, c, h // _HB, _HB, w)
    x2 = input2.reshape(b, h // _HB, _HB, w)
    out = pl.pallas_call(
        _resample_body,
        grid=(b, h // _HB),
        in_specs=[
            pl.BlockSpec((1, 1, _HB, w), lambda bi, hb: (bi, hb, 0, 0)),
            pl.BlockSpec((1, c, 1, _HB, w), lambda bi, hb: (bi, 0, hb, 0, 0)),
        ],
        out_specs=pl.BlockSpec(
            (1, c, 1, _HB, w), lambda bi, hb: (bi, 0, hb, 0, 0)),
        out_shape=jax.ShapeDtypeStruct((b, c, h // _HB, _HB, w), jnp.float32),
        scratch_shapes=[
            pltpu.VMEM((4, (_HB - 1) * _G + c + 1, 128), jnp.float32),
            pltpu.VMEM((4, (c - 1) * _GO + _HB + 1, 128), jnp.float32),
        ],
        compiler_params=pltpu.CompilerParams(
            dimension_semantics=("parallel", "arbitrary"),
            vmem_limit_bytes=56 * 1024 * 1024,
        ),
    )(x2, x1)
    return out.reshape(b, c, h, w)


def kernel(input1, input2):
    return _resample(input1, input2)


# speedup vs baseline: 4.2475x; 1.0209x over previous
import jax
import jax.numpy as jnp
import numpy as np
from jax.experimental import pallas as pl
from jax.experimental.pallas import tpu as pltpu
from jax.sharding import Mesh, NamedSharding, PartitionSpec as P

_B, _C, _H, _W = 4, 64, 256, 512
_HB = 32


_G = 65
_GO = _HB + 1
_HALF = _HB // 8


def _resample_body(x2_ref, x1_ref, o_ref, m_scr, o_scr):
    val = x1_ref[0, :, 0, :, :].reshape(_C * _HB, _W)
    for ci in range(_C):
        for half in range(_HALF):
            row0 = (ci * _HALF + half) * 8
            dst0 = half * 8 * _G + ci
            for wc in range(4):
                m_scr[wc, dst0:dst0 + 8 * _G:_G, :] = (
                    val[row0:row0 + 8, wc * 128:(wc + 1) * 128])
    disp = x2_ref[0, 0, :, :]
    iota_w = jax.lax.broadcasted_iota(
        jnp.int32, (_HB, _W), 1).astype(jnp.float32)
    x = iota_w + disp
    valid = (x >= 0.0) & (x <= float(_W - 1))
    xa = jnp.where(valid, x, -2.0)
    col = jax.lax.broadcasted_iota(
        jnp.int32, (_W, _W), 0).astype(jnp.float32)
    one = jnp.bfloat16(1.0)
    for hi in range(_HB):
        xr = xa[hi:hi + 1, :]
        t = (col - xr).astype(jnp.bfloat16)
        s = one - jnp.minimum(jnp.abs(t), one)
        base = hi * _G
        lhs = jnp.concatenate(
            [m_scr[wc, base:base + _C, :] for wc in range(4)],
            axis=-1).astype(jnp.bfloat16)
        res = jnp.dot(lhs, s, preferred_element_type=jnp.float32)
        for g in range(_C // 8):
            dst0 = 8 * g * _GO + hi
            for wc in range(4):
                o_scr[wc, dst0:dst0 + 8 * _GO:_GO, :] = (
                    res[8 * g:8 * g + 8, wc * 128:(wc + 1) * 128])
    for ci in range(_C):
        for wc in range(4):
            o_ref[0, ci, 0, :, wc * 128:(wc + 1) * 128] = (
                o_scr[wc, ci * _GO:ci * _GO + _HB, :])


def _resample(input1, input2):
    b, c, h, w = input1.shape
    x1 = input1.reshape(b, c, h // _HB, _HB, w)
    x2 = input2.reshape(b, h // _HB, _HB, w)
    out = pl.pallas_call(
        _resample_body,
        grid=(b, h // _HB),
        in_specs=[
            pl.BlockSpec((1, 1, _HB, w), lambda bi, hb: (bi, hb, 0, 0)),
            pl.BlockSpec((1, c, 1, _HB, w), lambda bi, hb: (bi, 0, hb, 0, 0)),
        ],
        out_specs=pl.BlockSpec(
            (1, c, 1, _HB, w), lambda bi, hb: (bi, 0, hb, 0, 0)),
        out_shape=jax.ShapeDtypeStruct((b, c, h // _HB, _HB, w), jnp.float32),
        scratch_shapes=[
            pltpu.VMEM((4, (_HB - 1) * _G + c + 1, 128), jnp.float32),
            pltpu.VMEM((4, (c - 1) * _GO + _HB + 1, 128), jnp.float32),
        ],
        compiler_params=pltpu.CompilerParams(
            dimension_semantics=("parallel", "arbitrary"),
            vmem_limit_bytes=56 * 1024 * 1024,
        ),
    )(x2, x1)
    return out.reshape(b, c, h, w)


def kernel(input1, input2):
    return _resample(input1, input2)


# speedup vs baseline: 4.2903x; 1.0101x over previous
import jax
import jax.numpy as jnp
import numpy as np
from jax.experimental import pallas as pl
from jax.experimental.pallas import tpu as pltpu
from jax.sharding import Mesh, NamedSharding, PartitionSpec as P

_B, _C, _H, _W = 4, 64, 256, 512
_HB = 64


_G = 65
_GO = _HB + 1
_HALF = _HB // 8


def _resample_body(x2_ref, x1_ref, o_ref, m_scr, o_scr):
    val = x1_ref[0, :, 0, :, :].reshape(_C * _HB, _W)
    for ci in range(_C):
        for half in range(_HALF):
            row0 = (ci * _HALF + half) * 8
            dst0 = half * 8 * _G + ci
            for wc in range(4):
                m_scr[wc, dst0:dst0 + 8 * _G:_G, :] = (
                    val[row0:row0 + 8, wc * 128:(wc + 1) * 128])
    disp = x2_ref[0, 0, :, :]
    iota_w = jax.lax.broadcasted_iota(
        jnp.int32, (_HB, _W), 1).astype(jnp.float32)
    x = iota_w + disp
    valid = (x >= 0.0) & (x <= float(_W - 1))
    xa = jnp.where(valid, x, -2.0)
    col = jax.lax.broadcasted_iota(
        jnp.int32, (_W, _W), 0).astype(jnp.float32)
    one = jnp.bfloat16(1.0)
    for hi in range(_HB):
        xr = xa[hi:hi + 1, :]
        t = (col - xr).astype(jnp.bfloat16)
        s = one - jnp.minimum(jnp.abs(t), one)
        base = hi * _G
        lhs = jnp.concatenate(
            [m_scr[wc, base:base + _C, :] for wc in range(4)],
            axis=-1).astype(jnp.bfloat16)
        res = jnp.dot(lhs, s, preferred_element_type=jnp.float32)
        for g in range(_C // 8):
            dst0 = 8 * g * _GO + hi
            for wc in range(4):
                o_scr[wc, dst0:dst0 + 8 * _GO:_GO, :] = (
                    res[8 * g:8 * g + 8, wc * 128:(wc + 1) * 128])
    for ci in range(_C):
        for wc in range(4):
            o_ref[0, ci, 0, :, wc * 128:(wc + 1) * 128] = (
                o_scr[wc, ci * _GO:ci * _GO + _HB, :])


def _resample(input1, input2):
    b, c, h, w = input1.shape
    x1 = input1.reshape(b, c, h // _HB, _HB, w)
    x2 = input2.reshape(b, h // _HB, _HB, w)
    out = pl.pallas_call(
        _resample_body,
        grid=(b, h // _HB),
        in_specs=[
            pl.BlockSpec((1, 1, _HB, w), lambda bi, hb: (bi, hb, 0, 0)),
            pl.BlockSpec((1, c, 1, _HB, w), lambda bi, hb: (bi, 0, hb, 0, 0)),
        ],
        out_specs=pl.BlockSpec(
            (1, c, 1, _HB, w), lambda bi, hb: (bi, 0, hb, 0, 0)),
        out_shape=jax.ShapeDtypeStruct((b, c, h // _HB, _HB, w), jnp.float32),
        scratch_shapes=[
            pltpu.VMEM((4, (_HB - 1) * _G + c + 1, 128), jnp.float32),
            pltpu.VMEM((4, (c - 1) * _GO + _HB + 1, 128), jnp.float32),
        ],
        compiler_params=pltpu.CompilerParams(
            dimension_semantics=("parallel", "arbitrary"),
            vmem_limit_bytes=56 * 1024 * 1024,
        ),
    )(x2, x1)
    return out.reshape(b, c, h, w)


def kernel(input1, input2):
    return _resample(input1, input2)
